# baseline (speedup 1.0000x reference)
import functools

import jax
import jax.numpy as jnp
import numpy as np
from jax import lax
from jax.experimental import pallas as pl
from jax.experimental.pallas import tpu as pltpu
from jax.experimental.pallas import tpu_sc as plsc

_N = 10000
_E_RAW = 320000
_E = _E_RAW + _N
_G = 16
_H = 64
_DSE = 32
_FM = _H + _DSE
_FC = _H
_PRED_H = 16
_NODE_FEAT = 4
_N_DIFF = 200

_NP = 10240
_EP = 344064
_EPH = _EP // 2
_WF_C = 80
_WF_M = 112
_WT_C = 48
_WT_M = 64
_MT_W = 80

_BE = 2048
_BN = 1024

_NC, _NS = 2, 16
_CH = _EPH // 128 // (_NC * _NS)
_NROW = _NP // _NS


def _pe_tab():
    pos = np.arange(_N_DIFF, dtype=np.float64)[:, None] + 1.0
    div = np.exp(np.arange(0, _DSE, 2, dtype=np.float64) * -(np.log(10000.0) / _DSE))
    pe = np.zeros((_N_DIFF, _DSE), dtype=np.float32)
    pe[:, 0::2] = np.sin(pos * div)
    pe[:, 1::2] = np.cos(pos * div)
    return jnp.asarray(pe)


def _silu(v):
    return v * (0.5 * jnp.tanh(0.5 * v) + 0.5)


def _full_spec(shape):
    nd = len(shape)
    return pl.BlockSpec(shape, lambda i: (0,) * nd)


def _sc_mesh():
    return plsc.VectorSubcoreMesh(core_axis_name="c", subcore_axis_name="s",
                                  num_cores=_NC, num_subcores=_NS)


_SC_PARAMS = pltpu.CompilerParams(use_tc_tiling_on_sc=False)


def _sc_gather(table, rowi, coli):
    w = table.shape[1]
    nb = 3

    def body(tab, ri, ci, outr, outc, idxr, idxc,
             bufr0, bufr1, bufr2, bufc0, bufc1, bufc2,
             sgr0, sgr1, sgr2, sgc0, sgc1, sgc2,
             swr0, swr1, swr2, swc0, swc1, swc2):
        bufr = (bufr0, bufr1, bufr2)
        bufc = (bufc0, bufc1, bufc2)
        sgr = (sgr0, sgr1, sgr2)
        sgc = (sgc0, sgc1, sgc2)
        swr = (swr0, swr1, swr2)
        swc = (swc0, swc1, swc2)
        c = lax.axis_index("c")
        s = lax.axis_index("s")
        wid = s * _NC + c
        base = wid * _CH
        pltpu.sync_copy(ri.at[wid], idxr)
        pltpu.sync_copy(ci.at[wid], idxc)

        def wr_desc(b, j):
            return (pltpu.make_async_copy(
                        bufr[b], outr.at[pl.ds((base + j) * 128, 128)], swr[b]),
                    pltpu.make_async_copy(
                        bufc[b], outc.at[pl.ds((base + j) * 128, 128)], swc[b]))

        def gd_desc(b, j):
            return (pltpu.make_async_copy(tab.at[idxr.at[j]], bufr[b], sgr[b]),
                    pltpu.make_async_copy(tab.at[idxc.at[j]], bufc[b], sgc[b]))

        def group(g, carry):
            for b in range(nb):
                j = g * nb + b
                pb = (b + nb - 1) % nb

                @pl.when(j >= nb)
                def _():
                    d1, d2 = wr_desc(b, j - nb)
                    d1.wait()
                    d2.wait()

                g1, g2 = gd_desc(b, j)
                g1.start()
                g2.start()

                @pl.when(j >= 1)
                def _():
                    p1, p2 = gd_desc(pb, j - 1)
                    p1.wait()
                    p2.wait()
                    e1, e2 = wr_desc(pb, j - 1)
                    e1.start()
                    e2.start()
            return carry

        lax.fori_loop(0, _CH // nb, group, 0)
        last = _CH - 1
        lb = last % nb
        p1, p2 = gd_desc(lb, last)
        p1.wait()
        p2.wait()
        e1, e2 = wr_desc(lb, last)
        e1.start()
        e2.start()
        for b in range(nb):
            j = _CH - nb + b
            d1, d2 = wr_desc(j % nb, j)
            d1.wait()
            d2.wait()

    return pl.kernel(
        body,
        out_type=[jax.ShapeDtypeStruct((_EPH, w), jnp.float32)] * 2,
        mesh=_sc_mesh(),
        scratch_types=(
            [pltpu.VMEM((_CH, 128), jnp.int32)] * 2
            + [pltpu.VMEM((128, w), jnp.float32)] * 6
            + [pltpu.SemaphoreType.DMA] * 12
        ),
        compiler_params=_SC_PARAMS,
    )(table, rowi, coli)


def _sc_scatter(mt, rowi, zeros_np):
    nb = 3

    def body(mt_h, ri, z_h, out_h, idx, buf0, buf1, buf2, acc_sh,
             sl0, sl1, sl2, sa0, sa1, sa2):
        buf = (buf0, buf1, buf2)
        sl = (sl0, sl1, sl2)
        sa = (sa0, sa1, sa2)
        c = lax.axis_index("c")
        s = lax.axis_index("s")
        wid = s * _NC + c
        base = wid * _CH
        pltpu.sync_copy(z_h.at[pl.ds(s * _NROW, _NROW)],
                        acc_sh.at[pl.ds(s * _NROW, _NROW)])
        pltpu.sync_copy(ri.at[wid], idx)
        plsc.subcore_barrier()

        def ld_desc(b, j):
            return pltpu.make_async_copy(
                mt_h.at[pl.ds((base + j) * 128, 128)], buf[b], sl[b])

        def add_desc(b, j):
            return pltpu.make_async_copy(buf[b], acc_sh.at[idx.at[j]], sa[b])

        def group(g, carry):
            for b in range(nb):
                j = g * nb + b
                pb = (b + nb - 1) % nb

                @pl.when(j >= nb)
                def _():
                    add_desc(b, j - nb).wait()

                ld_desc(b, j).start()

                @pl.when(j >= 1)
                def _():
                    ld_desc(pb, j - 1).wait()
                    pltpu.async_copy(buf[pb], acc_sh.at[idx.at[j - 1]],
                                     sa[pb], add=True)
            return carry

        lax.fori_loop(0, _CH // nb, group, 0)
        last = _CH - 1
        lb = last % nb
        ld_desc(lb, last).wait()
        pltpu.async_copy(buf[lb], acc_sh.at[idx.at[last]], sa[lb], add=True)
        for b in range(nb):
            j = _CH - nb + b
            add_desc(j % nb, j).wait()
        plsc.subcore_barrier()
        pltpu.sync_copy(acc_sh.at[pl.ds(s * _NROW, _NROW)],
                        out_h.at[c, pl.ds(s * _NROW, _NROW)])

    return pl.kernel(
        body,
        out_type=jax.ShapeDtypeStruct((2, _NP, _MT_W), jnp.float32),
        mesh=_sc_mesh(),
        scratch_types=(
            [pltpu.VMEM((_CH, 128), jnp.int32)]
            + [pltpu.VMEM((128, _MT_W), jnp.float32)] * 3
            + [pltpu.VMEM_SHARED((_NP, _MT_W), jnp.float32)]
            + [pltpu.SemaphoreType.DMA] * 6
        ),
        compiler_params=_SC_PARAMS,
    )(mt, rowi, zeros_np)


def _unpack2_bf16(v):
    uw = jax.lax.bitcast_convert_type(v, jnp.uint32)
    a = jax.lax.bitcast_convert_type(uw & jnp.uint32(0xFFFF0000), jnp.float32)
    b = jax.lax.bitcast_convert_type(uw << 16, jnp.float32)
    return a.astype(jnp.bfloat16), b.astype(jnp.bfloat16)


def _pack_bf16(v):
    h = v.shape[1] // 2
    r1 = v[:, :h].astype(jnp.bfloat16).astype(jnp.float32)
    r2 = v[:, h:].astype(jnp.bfloat16).astype(jnp.float32)
    u1 = jax.lax.bitcast_convert_type(r1, jnp.uint32)
    u2 = jax.lax.bitcast_convert_type(r2, jnp.uint32)
    w = (u1 & jnp.uint32(0xFFFF0000)) | (u2 >> 16)
    return jax.lax.bitcast_convert_type(w, jnp.float32)


def _edge_call(hrow, hcol, ea2, w1a, w1b, wr3, w1e, b1f, w2, b2, wc1, bc1,
               wc2, normalize, f):
    wf = hrow.shape[1]
    fp = f // 2
    grid = _EPH // _BE

    def body(hr_ref, hc_ref, ea_ref, w1a_ref, w1b_ref, wr3_ref, w1e_ref,
             b1f_ref, w2_ref, b2_ref, wc1_ref, bc1_ref, wc2_ref, out_ref):
        hr = hr_ref[...]
        hc = hc_ref[...]
        cd = hr[:, fp:fp + 3] - hc[:, fp:fp + 3]
        cdsq = cd * cd
        ra, rb = _unpack2_bf16(hr[:, :fp])
        ca, cb = _unpack2_bf16(hc[:, :fp])
        pre = (jnp.dot(ra, w1a_ref[0], preferred_element_type=jnp.float32)
               + jnp.dot(rb, w1a_ref[1], preferred_element_type=jnp.float32)
               + jnp.dot(ca, w1b_ref[0], preferred_element_type=jnp.float32)
               + jnp.dot(cb, w1b_ref[1], preferred_element_type=jnp.float32))
        pre = (pre
               + jnp.dot(cdsq, wr3_ref[...], preferred_element_type=jnp.float32)
               + jnp.dot(ea_ref[...], w1e_ref[...],
                         preferred_element_type=jnp.float32)
               + b1f_ref[...])
        h1 = _silu(pre)
        m = _silu(jnp.dot(h1.astype(jnp.bfloat16), w2_ref[...],
                          preferred_element_type=jnp.float32) + b2_ref[...])
        c1 = _silu(jnp.dot(m.astype(jnp.bfloat16), wc1_ref[...],
                           preferred_element_type=jnp.float32) + bc1_ref[...])
        cm = jnp.dot(c1.astype(jnp.bfloat16), wc2_ref[...],
                     preferred_element_type=jnp.float32)
        if normalize:
            radial = jnp.sum(cdsq, axis=1, keepdims=True)
            sc_ = cm / (jnp.sqrt(radial) + 1e-8)
        else:
            sc_ = cm
        out_ref[:, 0:_H] = m
        out_ref[:, _H:_H + 3] = cd * sc_
        out_ref[:, _H + 3:_H + 4] = jnp.ones((_BE, 1), jnp.float32)
        out_ref[:, _H + 4:_MT_W] = jnp.zeros((_BE, _MT_W - _H - 4), jnp.float32)

    return pl.pallas_call(
        body,
        grid=(grid,),
        in_specs=[
            pl.BlockSpec((_BE, wf), lambda i: (i, 0)),
            pl.BlockSpec((_BE, wf), lambda i: (i, 0)),
            pl.BlockSpec((_BE, 1), lambda i: (i, 0)),
            _full_spec(w1a.shape),
            _full_spec(w1b.shape),
            _full_spec(wr3.shape),
            _full_spec(w1e.shape),
            _full_spec(b1f.shape),
            _full_spec(w2.shape),
            _full_spec(b2.shape),
            _full_spec(wc1.shape),
            _full_spec(bc1.shape),
            _full_spec(wc2.shape),
        ],
        out_specs=pl.BlockSpec((_BE, _MT_W), lambda i: (i, 0)),
        out_shape=jax.ShapeDtypeStruct((_EPH, _MT_W), jnp.float32),
    )(hrow, hcol, ea2, w1a, w1b, wr3, w1e, b1f, w2, b2, wc1, bc1, wc2)


def _node_call(hx, acca, accb, batch2, w1h, w1a, b1, w2, b2, scales, biases,
               film, f):
    wf = hx.shape[1]
    wt = _WT_M if f == _FM else _WT_C
    grid = _NP // _BN

    def body(hx_ref, acc_ref, accb_ref, b_ref, w1h_ref, w1a_ref, b1_ref,
             w2_ref, b2_ref, s_ref, bi_ref, out_ref, tab_ref):
        hxv = hx_ref[...]
        h = hxv[:, :f]
        coord = hxv[:, f:f + 3]
        acc = (acc_ref[0] + acc_ref[1]) + (accb_ref[0] + accb_ref[1])
        agg = acc[:, :_H]
        tr = acc[:, _H:_H + 3]
        cnt = acc[:, _H + 3:_H + 4]
        coord2 = coord + tr / jnp.maximum(cnt, 1.0)
        pre = (jnp.dot(h, w1h_ref[...], preferred_element_type=jnp.float32)
               + jnp.dot(agg, w1a_ref[...], preferred_element_type=jnp.float32)
               + b1_ref[...])
        hmid = _silu(pre)
        h2 = h + jnp.dot(hmid, w2_ref[...],
                         preferred_element_type=jnp.float32) + b2_ref[...]
        if film:
            bb = b_ref[...]
            oh = (bb == lax.broadcasted_iota(jnp.int32, (_BN, _G), 1)
                  ).astype(jnp.float32)
            h2 = (jnp.dot(oh, s_ref[...], preferred_element_type=jnp.float32)
                  * h2
                  + jnp.dot(oh, bi_ref[...], preferred_element_type=jnp.float32))
        pad = jnp.zeros((_BN, wf - f - 3), jnp.float32)
        out_ref[...] = jnp.concatenate([h2, coord2, pad], axis=1)
        tpad = jnp.zeros((_BN, wt - f // 2 - 3), jnp.float32)
        tab_ref[...] = jnp.concatenate([_pack_bf16(h2), coord2, tpad], axis=1)

    return pl.pallas_call(
        body,
        grid=(grid,),
        in_specs=[
            pl.BlockSpec((_BN, wf), lambda i: (i, 0)),
            pl.BlockSpec((2, _BN, _MT_W), lambda i: (0, i, 0)),
            pl.BlockSpec((2, _BN, _MT_W), lambda i: (0, i, 0)),
            pl.BlockSpec((_BN, 1), lambda i: (i, 0)),
            _full_spec(w1h.shape),
            _full_spec(w1a.shape),
            _full_spec(b1.shape),
            _full_spec(w2.shape),
            _full_spec(b2.shape),
            _full_spec(scales.shape),
            _full_spec(biases.shape),
        ],
        out_specs=[pl.BlockSpec((_BN, wf), lambda i: (i, 0)),
                   pl.BlockSpec((_BN, wt), lambda i: (i, 0))],
        out_shape=[jax.ShapeDtypeStruct((_NP, wf), jnp.float32),
                   jax.ShapeDtypeStruct((_NP, wt), jnp.float32)],
    )(hx, acca, accb, batch2, w1h, w1a, b1, w2, b2, scales, biases)


def _cond_init_call(cond2, coord2, wci, bci):
    grid = _NP // _BN

    def body(c_ref, xy_ref, w_ref, b_ref, out_ref, tab_ref):
        h = jnp.dot(c_ref[...], w_ref[...],
                    preferred_element_type=jnp.float32) + b_ref[...]
        xy = xy_ref[...]
        pad = jnp.zeros((_BN, _WF_C - _FC - 3), jnp.float32)
        out_ref[...] = jnp.concatenate([h, xy, pad], axis=1)
        tpad = jnp.zeros((_BN, _WT_C - _FC // 2 - 3), jnp.float32)
        tab_ref[...] = jnp.concatenate([_pack_bf16(h), xy, tpad], axis=1)

    return pl.pallas_call(
        body,
        grid=(grid,),
        in_specs=[
            pl.BlockSpec((_BN, cond2.shape[1]), lambda i: (i, 0)),
            pl.BlockSpec((_BN, 3), lambda i: (i, 0)),
            _full_spec(wci.shape),
            _full_spec(bci.shape),
        ],
        out_specs=[pl.BlockSpec((_BN, _WF_C), lambda i: (i, 0)),
                   pl.BlockSpec((_BN, _WT_C), lambda i: (i, 0))],
        out_shape=[jax.ShapeDtypeStruct((_NP, _WF_C), jnp.float32),
                   jax.ShapeDtypeStruct((_NP, _WT_C), jnp.float32)],
    )(cond2, coord2, wci, bci)


def _t_call(ts2, pe, wd1, bd1, wd2, bd2):
    def body(ts_ref, pe_ref, w1_ref, b1_ref, w2_ref, b2_ref, out_ref):
        oh = (ts_ref[...] == lax.broadcasted_iota(jnp.int32, (_G, _N_DIFF), 1)
              ).astype(jnp.float32)
        t = jnp.dot(oh, pe_ref[...], preferred_element_type=jnp.float32)
        v = jnp.dot(t, w1_ref[...], preferred_element_type=jnp.float32) + b1_ref[...]
        u = v * jnp.tanh(jax.nn.softplus(v))
        out_ref[...] = jnp.dot(u, w2_ref[...],
                               preferred_element_type=jnp.float32) + b2_ref[...]

    return pl.pallas_call(
        body,
        grid=(1,),
        in_specs=[_full_spec(ts2.shape), _full_spec(pe.shape),
                  _full_spec(wd1.shape), _full_spec(bd1.shape),
                  _full_spec(wd2.shape), _full_spec(bd2.shape)],
        out_specs=_full_spec((_G, _DSE)),
        out_shape=jax.ShapeDtypeStruct((_G, _DSE), jnp.float32),
    )(ts2, pe, wd1, bd1, wd2, bd2)


def _main_init_call(x2, batch2, coord2, t2, s0, b0, wne, bne):
    grid = _NP // _BN

    def body(x_ref, b_ref, xy_ref, t_ref, s_ref, bi_ref, w_ref, bn_ref,
             out_ref, tab_ref):
        oh = (b_ref[...] == lax.broadcasted_iota(jnp.int32, (_BN, _G), 1)
              ).astype(jnp.float32)
        temb = jnp.dot(oh, t_ref[...], preferred_element_type=jnp.float32)
        hn = jnp.dot(x_ref[...], w_ref[...],
                     preferred_element_type=jnp.float32) + bn_ref[...]
        h = jnp.concatenate([hn, temb], axis=1)
        h = (jnp.dot(oh, s_ref[...], preferred_element_type=jnp.float32) * h
             + jnp.dot(oh, bi_ref[...], preferred_element_type=jnp.float32))
        xy = xy_ref[...]
        pad = jnp.zeros((_BN, _WF_M - _FM - 3), jnp.float32)
        out_ref[...] = jnp.concatenate([h, xy, pad], axis=1)
        tpad = jnp.zeros((_BN, _WT_M - _FM // 2 - 3), jnp.float32)
        tab_ref[...] = jnp.concatenate([_pack_bf16(h), xy, tpad], axis=1)

    return pl.pallas_call(
        body,
        grid=(grid,),
        in_specs=[
            pl.BlockSpec((_BN, x2.shape[1]), lambda i: (i, 0)),
            pl.BlockSpec((_BN, 1), lambda i: (i, 0)),
            pl.BlockSpec((_BN, 3), lambda i: (i, 0)),
            _full_spec(t2.shape),
            _full_spec(s0.shape),
            _full_spec(b0.shape),
            _full_spec(wne.shape),
            _full_spec(bne.shape),
        ],
        out_specs=[pl.BlockSpec((_BN, _WF_M), lambda i: (i, 0)),
                   pl.BlockSpec((_BN, _WT_M), lambda i: (i, 0))],
        out_shape=[jax.ShapeDtypeStruct((_NP, _WF_M), jnp.float32),
                   jax.ShapeDtypeStruct((_NP, _WT_M), jnp.float32)],
    )(x2, batch2, coord2, t2, s0, b0, wne, bne)


def _cond_final_call(hcx, batch2, wco, bco, wfc, bfc):
    def body(hx_ref, b_ref, wco_ref, bco_ref, wfc_ref, bfc_ref, out_ref):
        hc = hx_ref[...][:, :_FC]
        h = jnp.dot(hc, wco_ref[...],
                    preferred_element_type=jnp.float32) + bco_ref[...]
        oh = (b_ref[...] == lax.broadcasted_iota(jnp.int32, (_NP, _G), 1)
              ).astype(jnp.float32)
        seg = lax.dot_general(oh, h, (((0,), (0,)), ((), ())),
                              preferred_element_type=jnp.float32)
        ones = jnp.ones((_NP, 1), jnp.float32)
        cnt = lax.dot_general(oh, ones, (((0,), (0,)), ((), ())),
                              preferred_element_type=jnp.float32)
        g = seg / jnp.maximum(cnt, 1.0)
        out_ref[...] = jnp.dot(g, wfc_ref[...],
                               preferred_element_type=jnp.float32) + bfc_ref[...]

    return pl.pallas_call(
        body,
        grid=(1,),
        in_specs=[_full_spec(hcx.shape), _full_spec(batch2.shape),
                  _full_spec(wco.shape), _full_spec(bco.shape),
                  _full_spec(wfc.shape), _full_spec(bfc.shape)],
        out_specs=_full_spec((_G, wfc.shape[1])),
        out_shape=jax.ShapeDtypeStruct((_G, wfc.shape[1]), jnp.float32),
    )(hcx, batch2, wco, bco, wfc, bfc)


def _pred_call(hx, w1, b1, w2, b2, w3, b3):
    grid = _NP // _BN

    def body(hx_ref, w1_ref, b1_ref, w2_ref, b2_ref, w3_ref, b3_ref, out_ref):
        h = hx_ref[...][:, :_FM]
        o = jax.nn.relu(jnp.dot(h, w1_ref[...],
                                preferred_element_type=jnp.float32) + b1_ref[...])
        o = jax.nn.relu(jnp.dot(o, w2_ref[...],
                                preferred_element_type=jnp.float32) + b2_ref[...])
        out_ref[...] = jnp.dot(o, w3_ref[...],
                               preferred_element_type=jnp.float32) + b3_ref[...]

    return pl.pallas_call(
        body,
        grid=(grid,),
        in_specs=[
            pl.BlockSpec((_BN, _WF_M), lambda i: (i, 0)),
            _full_spec(w1.shape), _full_spec(b1.shape),
            _full_spec(w2.shape), _full_spec(b2.shape),
            _full_spec(w3.shape), _full_spec(b3.shape),
        ],
        out_specs=pl.BlockSpec((_BN, w3.shape[1]), lambda i: (i, 0)),
        out_shape=jax.ShapeDtypeStruct((_NP, w3.shape[1]), jnp.float32),
    )(hx, w1, b1, w2, b2, w3, b3)


def _gcl_edge_phase(tab, rowih, colih, ea2h, zeros_np, w, normalize, f):
    accs = []
    gathered = [_sc_gather(tab, rowih[hf], colih[hf]) for hf in range(2)]
    mts = [_edge_call(gathered[hf][0], gathered[hf][1], ea2h[hf],
                      w["w1a"], w["w1b"], w["wr3"], w["w1e"], w["b1f"],
                      w["w2"], w["b2"], w["wc1"], w["bc1"], w["wc2"],
                      normalize=normalize, f=f) for hf in range(2)]
    for hf in range(2):
        accs.append(_sc_scatter(mts[hf], rowih[hf], zeros_np))
    return accs


def _fold_gcl(p, f, wf, we=None, be=None):
    bf16 = jnp.bfloat16
    w1 = p["edge_mlp"][0]["W"]
    b1 = p["edge_mlp"][0]["b"]
    fp = f // 2
    w1a = jnp.stack([w1[:fp], w1[fp:f]]).astype(bf16)
    w1b = jnp.stack([w1[f:f + fp], w1[f + fp:2 * f]]).astype(bf16)
    w1r = w1[2 * f:2 * f + 1]
    if we is None:
        w1e = w1[2 * f + 1:2 * f + 2]
        b1f = b1[None, :]
    else:
        w1tail = w1[2 * f + 1:]
        w1e = we @ w1tail
        b1f = (b1 + be @ w1tail)[None, :]
    wr3 = jnp.concatenate([w1r, w1r, w1r], axis=0)
    w2 = p["edge_mlp"][1]["W"].astype(bf16)
    b2 = p["edge_mlp"][1]["b"][None, :]
    wc1 = p["coord_mlp"][0]["W"].astype(bf16)
    bc1 = p["coord_mlp"][0]["b"][None, :]
    wc2 = p["coord_mlp"][1]["W"].astype(bf16)
    wn1 = p["node_mlp"][0]["W"]
    w1h = wn1[:f]
    w1ag = wn1[f:]
    bn1 = p["node_mlp"][0]["b"][None, :]
    wn2 = p["node_mlp"][1]["W"]
    bn2 = p["node_mlp"][1]["b"][None, :]
    return dict(w1a=w1a, w1b=w1b, wr3=wr3, w1e=w1e, b1f=b1f, w2=w2, b2=b2,
                wc1=wc1, bc1=bc1, wc2=wc2, w1h=w1h, w1ag=w1ag, bn1=bn1,
                wn2=wn2, bn2=bn2)


def kernel(x, edge_index, edge_attr, x_coord, cond, timesteps, batch, params):
    f32 = jnp.float32
    i32 = jnp.int32

    x2 = jnp.zeros((_NP, _PRED_H * _NODE_FEAT), f32).at[:_N].set(
        x.reshape(_N, -1))
    loops = jnp.arange(_N, dtype=i32)
    row = jnp.concatenate([edge_index[0].astype(i32), loops])
    col = jnp.concatenate([edge_index[1].astype(i32), loops])
    ea = jnp.concatenate([edge_attr.astype(f32), jnp.zeros((_N,), f32)])
    rowi = jnp.full((_EP,), _NP - 1, i32).at[:_E].set(row).reshape(
        2, _NC * _NS, _CH, 128)
    coli = jnp.full((_EP,), _NP - 1, i32).at[:_E].set(col).reshape(
        2, _NC * _NS, _CH, 128)
    ea2 = jnp.zeros((_EP, 1), f32).at[:_E, 0].set(ea)
    ea2h = (ea2[:_EPH], ea2[_EPH:])
    rowih = (rowi[0], rowi[1])
    colih = (coli[0], coli[1])
    batch2 = jnp.full((_NP, 1), _G, i32).at[:_N, 0].set(batch.astype(i32))
    coord2 = jnp.zeros((_NP, 3), f32).at[:_N].set(x_coord.astype(f32))
    cond2 = jnp.zeros((_NP, cond.shape[1]), f32).at[:_N].set(cond)
    zeros_np = jnp.zeros((_NP, _MT_W), f32)
    ts2 = timesteps.astype(i32).reshape(_G, 1)

    t2 = _t_call(ts2, _pe_tab(),
                 params["dse1"]["W"], params["dse1"]["b"][None, :],
                 params["dse2"]["W"], params["dse2"]["b"][None, :])

    hcx, tabc = _cond_init_call(cond2, coord2, params["cond_emb_in"]["W"],
                                params["cond_emb_in"]["b"][None, :])
    for p in params["cond_gcl"]:
        w = _fold_gcl(p, _FC, _WF_C)
        accs = _gcl_edge_phase(tabc, rowih, colih, ea2h, zeros_np, w,
                               normalize=False, f=_FC)
        dummy = jnp.zeros((_G, _FC), f32)
        hcx, tabc = _node_call(hcx, accs[0], accs[1], batch2, w["w1h"],
                               w["w1ag"], w["bn1"], w["wn2"], w["bn2"],
                               dummy, dummy, film=False, f=_FC)

    emb = _cond_final_call(hcx, batch2, params["cond_emb_out"]["W"],
                           params["cond_emb_out"]["b"][None, :],
                           params["cond_fc"]["W"],
                           params["cond_fc"]["b"][None, :])
    er = emb.reshape(5, _G, 2, _FM)
    scl = er[:, :, 0]
    bia = er[:, :, 1]

    hx, tabm = _main_init_call(x2, batch2, coord2, t2, scl[0], bia[0],
                               params["node_emb"]["W"],
                               params["node_emb"]["b"][None, :])
    we = params["edge_emb"]["W"]
    be = params["edge_emb"]["b"]
    for l in range(5):
        w = _fold_gcl(params["layers"][l], _FM, _WF_M, we=we, be=be)
        accs = _gcl_edge_phase(tabm, rowih, colih, ea2h, zeros_np, w,
                               normalize=True, f=_FM)
        film = l < 4
        s_l = scl[l + 1] if film else jnp.zeros((_G, _FM), f32)
        b_l = bia[l + 1] if film else jnp.zeros((_G, _FM), f32)
        hx, tabm = _node_call(hx, accs[0], accs[1], batch2, w["w1h"],
                              w["w1ag"], w["bn1"], w["wn2"], w["bn2"],
                              s_l, b_l, film=film, f=_FM)

    pred = _pred_call(hx, params["pred1"]["W"], params["pred1"]["b"][None, :],
                      params["pred2"]["W"], params["pred2"]["b"][None, :],
                      params["pred3"]["W"], params["pred3"]["b"][None, :])
    node_pred = pred[:_N].reshape(_N, _PRED_H, _NODE_FEAT)
    x_v = hx[:_N, _FM:_FM + 3]
    return node_pred, x_v


# speedup vs baseline: 1.4478x; 1.4478x over previous
import functools

import jax
import jax.numpy as jnp
import numpy as np
from jax import lax
from jax.experimental import pallas as pl
from jax.experimental.pallas import tpu as pltpu
from jax.experimental.pallas import tpu_sc as plsc

_N = 10000
_E_RAW = 320000
_E = _E_RAW + _N
_G = 16
_H = 64
_DSE = 32
_FM = _H + _DSE
_FC = _H
_PRED_H = 16
_NODE_FEAT = 4
_N_DIFF = 200

_NP = 10240
_EP = 331776
_EPH = _EP
_WF_C = 80
_WF_M = 112
_WT_C = 48
_WT_M = 64
_MT_W = 80

_BE = 2048
_BN = 1024

_NC, _NS = 2, 16
_CH = _EPH // 128 // (_NC * _NS)
_NROW = _NP // _NS


def _pe_tab():
    pos = np.arange(_N_DIFF, dtype=np.float64)[:, None] + 1.0
    div = np.exp(np.arange(0, _DSE, 2, dtype=np.float64) * -(np.log(10000.0) / _DSE))
    pe = np.zeros((_N_DIFF, _DSE), dtype=np.float32)
    pe[:, 0::2] = np.sin(pos * div)
    pe[:, 1::2] = np.cos(pos * div)
    return jnp.asarray(pe)


def _silu(v):
    return v * (0.5 * jnp.tanh(0.5 * v) + 0.5)


def _full_spec(shape):
    nd = len(shape)
    return pl.BlockSpec(shape, lambda i: (0,) * nd)


def _sc_mesh():
    return plsc.VectorSubcoreMesh(core_axis_name="c", subcore_axis_name="s",
                                  num_cores=_NC, num_subcores=_NS)


_SC_PARAMS = pltpu.CompilerParams(use_tc_tiling_on_sc=False)


def _sc_gather(table, rowi, coli):
    w = table.shape[1]
    nb = 3

    def body(tab, ri, ci, outr, outc, idxr, idxc,
             bufr0, bufr1, bufr2, bufc0, bufc1, bufc2,
             sgr0, sgr1, sgr2, sgc0, sgc1, sgc2,
             swr0, swr1, swr2, swc0, swc1, swc2):
        bufr = (bufr0, bufr1, bufr2)
        bufc = (bufc0, bufc1, bufc2)
        sgr = (sgr0, sgr1, sgr2)
        sgc = (sgc0, sgc1, sgc2)
        swr = (swr0, swr1, swr2)
        swc = (swc0, swc1, swc2)
        c = lax.axis_index("c")
        s = lax.axis_index("s")
        wid = s * _NC + c
        base = wid * _CH
        pltpu.sync_copy(ri.at[wid], idxr)
        pltpu.sync_copy(ci.at[wid], idxc)

        def wr_desc(b, j):
            return (pltpu.make_async_copy(
                        bufr[b], outr.at[pl.ds((base + j) * 128, 128)], swr[b]),
                    pltpu.make_async_copy(
                        bufc[b], outc.at[pl.ds((base + j) * 128, 128)], swc[b]))

        def gd_desc(b, j):
            return (pltpu.make_async_copy(tab.at[idxr.at[j]], bufr[b], sgr[b]),
                    pltpu.make_async_copy(tab.at[idxc.at[j]], bufc[b], sgc[b]))

        def group(g, carry):
            for b in range(nb):
                j = g * nb + b
                pb = (b + nb - 1) % nb

                @pl.when(j >= nb)
                def _():
                    d1, d2 = wr_desc(b, j - nb)
                    d1.wait()
                    d2.wait()

                g1, g2 = gd_desc(b, j)
                g1.start()
                g2.start()

                @pl.when(j >= 1)
                def _():
                    p1, p2 = gd_desc(pb, j - 1)
                    p1.wait()
                    p2.wait()
                    e1, e2 = wr_desc(pb, j - 1)
                    e1.start()
                    e2.start()
            return carry

        lax.fori_loop(0, _CH // nb, group, 0)
        last = _CH - 1
        lb = last % nb
        p1, p2 = gd_desc(lb, last)
        p1.wait()
        p2.wait()
        e1, e2 = wr_desc(lb, last)
        e1.start()
        e2.start()
        for b in range(nb):
            j = _CH - nb + b
            d1, d2 = wr_desc(j % nb, j)
            d1.wait()
            d2.wait()

    return pl.kernel(
        body,
        out_type=[jax.ShapeDtypeStruct((_EPH, w), jnp.float32)] * 2,
        mesh=_sc_mesh(),
        scratch_types=(
            [pltpu.VMEM((_CH, 128), jnp.int32)] * 2
            + [pltpu.VMEM((128, w), jnp.float32)] * 6
            + [pltpu.SemaphoreType.DMA] * 12
        ),
        compiler_params=_SC_PARAMS,
    )(table, rowi, coli)


def _sc_scatter(mt, rowi, zeros_np):
    nb = 3

    def body(mt_h, ri, z_h, out_h, idx, buf0, buf1, buf2, acc_sh,
             sl0, sl1, sl2, sa0, sa1, sa2):
        buf = (buf0, buf1, buf2)
        sl = (sl0, sl1, sl2)
        sa = (sa0, sa1, sa2)
        c = lax.axis_index("c")
        s = lax.axis_index("s")
        wid = s * _NC + c
        base = wid * _CH
        pltpu.sync_copy(z_h.at[pl.ds(s * _NROW, _NROW)],
                        acc_sh.at[pl.ds(s * _NROW, _NROW)])
        pltpu.sync_copy(ri.at[wid], idx)
        plsc.subcore_barrier()

        def ld_desc(b, j):
            return pltpu.make_async_copy(
                mt_h.at[pl.ds((base + j) * 128, 128)], buf[b], sl[b])

        def add_desc(b, j):
            return pltpu.make_async_copy(buf[b], acc_sh.at[idx.at[j]], sa[b])

        def group(g, carry):
            for b in range(nb):
                j = g * nb + b
                pb = (b + nb - 1) % nb

                @pl.when(j >= nb)
                def _():
                    add_desc(b, j - nb).wait()

                ld_desc(b, j).start()

                @pl.when(j >= 1)
                def _():
                    ld_desc(pb, j - 1).wait()
                    pltpu.async_copy(buf[pb], acc_sh.at[idx.at[j - 1]],
                                     sa[pb], add=True)
            return carry

        lax.fori_loop(0, _CH // nb, group, 0)
        last = _CH - 1
        lb = last % nb
        ld_desc(lb, last).wait()
        pltpu.async_copy(buf[lb], acc_sh.at[idx.at[last]], sa[lb], add=True)
        for b in range(nb):
            j = _CH - nb + b
            add_desc(j % nb, j).wait()
        plsc.subcore_barrier()
        pltpu.sync_copy(acc_sh.at[pl.ds(s * _NROW, _NROW)],
                        out_h.at[c, pl.ds(s * _NROW, _NROW)])

    return pl.kernel(
        body,
        out_type=jax.ShapeDtypeStruct((2, _NP, _MT_W), jnp.float32),
        mesh=_sc_mesh(),
        scratch_types=(
            [pltpu.VMEM((_CH, 128), jnp.int32)]
            + [pltpu.VMEM((128, _MT_W), jnp.float32)] * 3
            + [pltpu.VMEM_SHARED((_NP, _MT_W), jnp.float32)]
            + [pltpu.SemaphoreType.DMA] * 6
        ),
        compiler_params=_SC_PARAMS,
    )(mt, rowi, zeros_np)


def _unpack2_bf16(v):
    uw = jax.lax.bitcast_convert_type(v, jnp.uint32)
    a = jax.lax.bitcast_convert_type(uw & jnp.uint32(0xFFFF0000), jnp.float32)
    b = jax.lax.bitcast_convert_type(uw << 16, jnp.float32)
    return a.astype(jnp.bfloat16), b.astype(jnp.bfloat16)


def _pack_bf16(v):
    h = v.shape[1] // 2
    r1 = v[:, :h].astype(jnp.bfloat16).astype(jnp.float32)
    r2 = v[:, h:].astype(jnp.bfloat16).astype(jnp.float32)
    u1 = jax.lax.bitcast_convert_type(r1, jnp.uint32)
    u2 = jax.lax.bitcast_convert_type(r2, jnp.uint32)
    w = (u1 & jnp.uint32(0xFFFF0000)) | (u2 >> 16)
    return jax.lax.bitcast_convert_type(w, jnp.float32)


def _edge_call(hrow, hcol, ea2, w1a, w1b, wr3, w1e, b1f, w2, b2, wc1, bc1,
               wc2, normalize, f):
    wf = hrow.shape[1]
    fp = f // 2
    grid = _EPH // _BE

    def body(hr_ref, hc_ref, ea_ref, w1a_ref, w1b_ref, wr3_ref, w1e_ref,
             b1f_ref, w2_ref, b2_ref, wc1_ref, bc1_ref, wc2_ref, out_ref):
        hr = hr_ref[...]
        hc = hc_ref[...]
        cd = hr[:, fp:fp + 3] - hc[:, fp:fp + 3]
        cdsq = cd * cd
        ra, rb = _unpack2_bf16(hr[:, :fp])
        ca, cb = _unpack2_bf16(hc[:, :fp])
        pre = (jnp.dot(ra, w1a_ref[0], preferred_element_type=jnp.float32)
               + jnp.dot(rb, w1a_ref[1], preferred_element_type=jnp.float32)
               + jnp.dot(ca, w1b_ref[0], preferred_element_type=jnp.float32)
               + jnp.dot(cb, w1b_ref[1], preferred_element_type=jnp.float32))
        radial = jnp.sum(cdsq, axis=1, keepdims=True)
        pre = (pre + radial * wr3_ref[0:1] + ea_ref[...] * w1e_ref[...]
               + b1f_ref[...])
        h1 = _silu(pre)
        m = _silu(jnp.dot(h1.astype(jnp.bfloat16), w2_ref[...],
                          preferred_element_type=jnp.float32) + b2_ref[...])
        c1 = _silu(jnp.dot(m.astype(jnp.bfloat16), wc1_ref[...],
                           preferred_element_type=jnp.float32) + bc1_ref[...])
        cm = jnp.dot(c1.astype(jnp.bfloat16), wc2_ref[...],
                     preferred_element_type=jnp.float32)
        if normalize:
            sc_ = cm / (jnp.sqrt(radial) + 1e-8)
        else:
            sc_ = cm
        ones = jnp.ones((_BE, 1), jnp.float32)
        zer = jnp.zeros((_BE, _MT_W - _H - 4), jnp.float32)
        out_ref[...] = jnp.concatenate([m, cd * sc_, ones, zer], axis=1)

    return pl.pallas_call(
        body,
        grid=(grid,),
        in_specs=[
            pl.BlockSpec((_BE, wf), lambda i: (i, 0)),
            pl.BlockSpec((_BE, wf), lambda i: (i, 0)),
            pl.BlockSpec((_BE, 1), lambda i: (i, 0)),
            _full_spec(w1a.shape),
            _full_spec(w1b.shape),
            _full_spec(wr3.shape),
            _full_spec(w1e.shape),
            _full_spec(b1f.shape),
            _full_spec(w2.shape),
            _full_spec(b2.shape),
            _full_spec(wc1.shape),
            _full_spec(bc1.shape),
            _full_spec(wc2.shape),
        ],
        out_specs=pl.BlockSpec((_BE, _MT_W), lambda i: (i, 0)),
        out_shape=jax.ShapeDtypeStruct((_EPH, _MT_W), jnp.float32),
    )(hrow, hcol, ea2, w1a, w1b, wr3, w1e, b1f, w2, b2, wc1, bc1, wc2)


def _node_call(hx, acca, batch2, w1h, w1a, b1, w2, b2, scales, biases,
               film, f):
    wf = hx.shape[1]
    wt = _WT_M if f == _FM else _WT_C
    grid = _NP // _BN

    def body(hx_ref, acc_ref, b_ref, w1h_ref, w1a_ref, b1_ref,
             w2_ref, b2_ref, s_ref, bi_ref, out_ref, tab_ref):
        hxv = hx_ref[...]
        h = hxv[:, :f]
        coord = hxv[:, f:f + 3]
        acc = acc_ref[0] + acc_ref[1]
        agg = acc[:, :_H]
        tr = acc[:, _H:_H + 3]
        cnt = acc[:, _H + 3:_H + 4]
        coord2 = coord + tr / jnp.maximum(cnt, 1.0)
        pre = (jnp.dot(h, w1h_ref[...], preferred_element_type=jnp.float32)
               + jnp.dot(agg, w1a_ref[...], preferred_element_type=jnp.float32)
               + b1_ref[...])
        hmid = _silu(pre)
        h2 = h + jnp.dot(hmid, w2_ref[...],
                         preferred_element_type=jnp.float32) + b2_ref[...]
        if film:
            bb = b_ref[...]
            oh = (bb == lax.broadcasted_iota(jnp.int32, (_BN, _G), 1)
                  ).astype(jnp.float32)
            h2 = (jnp.dot(oh, s_ref[...], preferred_element_type=jnp.float32)
                  * h2
                  + jnp.dot(oh, bi_ref[...], preferred_element_type=jnp.float32))
        pad = jnp.zeros((_BN, wf - f - 3), jnp.float32)
        out_ref[...] = jnp.concatenate([h2, coord2, pad], axis=1)
        tpad = jnp.zeros((_BN, wt - f // 2 - 3), jnp.float32)
        tab_ref[...] = jnp.concatenate([_pack_bf16(h2), coord2, tpad], axis=1)

    return pl.pallas_call(
        body,
        grid=(grid,),
        in_specs=[
            pl.BlockSpec((_BN, wf), lambda i: (i, 0)),
            pl.BlockSpec((2, _BN, _MT_W), lambda i: (0, i, 0)),
            pl.BlockSpec((_BN, 1), lambda i: (i, 0)),
            _full_spec(w1h.shape),
            _full_spec(w1a.shape),
            _full_spec(b1.shape),
            _full_spec(w2.shape),
            _full_spec(b2.shape),
            _full_spec(scales.shape),
            _full_spec(biases.shape),
        ],
        out_specs=[pl.BlockSpec((_BN, wf), lambda i: (i, 0)),
                   pl.BlockSpec((_BN, wt), lambda i: (i, 0))],
        out_shape=[jax.ShapeDtypeStruct((_NP, wf), jnp.float32),
                   jax.ShapeDtypeStruct((_NP, wt), jnp.float32)],
    )(hx, acca, batch2, w1h, w1a, b1, w2, b2, scales, biases)


def _cond_init_call(cond2, coord2, wci, bci):
    grid = _NP // _BN

    def body(c_ref, xy_ref, w_ref, b_ref, out_ref, tab_ref):
        h = jnp.dot(c_ref[...], w_ref[...],
                    preferred_element_type=jnp.float32) + b_ref[...]
        xy = xy_ref[...]
        pad = jnp.zeros((_BN, _WF_C - _FC - 3), jnp.float32)
        out_ref[...] = jnp.concatenate([h, xy, pad], axis=1)
        tpad = jnp.zeros((_BN, _WT_C - _FC // 2 - 3), jnp.float32)
        tab_ref[...] = jnp.concatenate([_pack_bf16(h), xy, tpad], axis=1)

    return pl.pallas_call(
        body,
        grid=(grid,),
        in_specs=[
            pl.BlockSpec((_BN, cond2.shape[1]), lambda i: (i, 0)),
            pl.BlockSpec((_BN, 3), lambda i: (i, 0)),
            _full_spec(wci.shape),
            _full_spec(bci.shape),
        ],
        out_specs=[pl.BlockSpec((_BN, _WF_C), lambda i: (i, 0)),
                   pl.BlockSpec((_BN, _WT_C), lambda i: (i, 0))],
        out_shape=[jax.ShapeDtypeStruct((_NP, _WF_C), jnp.float32),
                   jax.ShapeDtypeStruct((_NP, _WT_C), jnp.float32)],
    )(cond2, coord2, wci, bci)


def _t_call(ts2, pe, wd1, bd1, wd2, bd2):
    def body(ts_ref, pe_ref, w1_ref, b1_ref, w2_ref, b2_ref, out_ref):
        oh = (ts_ref[...] == lax.broadcasted_iota(jnp.int32, (_G, _N_DIFF), 1)
              ).astype(jnp.float32)
        t = jnp.dot(oh, pe_ref[...], preferred_element_type=jnp.float32)
        v = jnp.dot(t, w1_ref[...], preferred_element_type=jnp.float32) + b1_ref[...]
        u = v * jnp.tanh(jax.nn.softplus(v))
        out_ref[...] = jnp.dot(u, w2_ref[...],
                               preferred_element_type=jnp.float32) + b2_ref[...]

    return pl.pallas_call(
        body,
        grid=(1,),
        in_specs=[_full_spec(ts2.shape), _full_spec(pe.shape),
                  _full_spec(wd1.shape), _full_spec(bd1.shape),
                  _full_spec(wd2.shape), _full_spec(bd2.shape)],
        out_specs=_full_spec((_G, _DSE)),
        out_shape=jax.ShapeDtypeStruct((_G, _DSE), jnp.float32),
    )(ts2, pe, wd1, bd1, wd2, bd2)


def _main_init_call(x2, batch2, coord2, t2, s0, b0, wne, bne):
    grid = _NP // _BN

    def body(x_ref, b_ref, xy_ref, t_ref, s_ref, bi_ref, w_ref, bn_ref,
             out_ref, tab_ref):
        oh = (b_ref[...] == lax.broadcasted_iota(jnp.int32, (_BN, _G), 1)
              ).astype(jnp.float32)
        temb = jnp.dot(oh, t_ref[...], preferred_element_type=jnp.float32)
        hn = jnp.dot(x_ref[...], w_ref[...],
                     preferred_element_type=jnp.float32) + bn_ref[...]
        h = jnp.concatenate([hn, temb], axis=1)
        h = (jnp.dot(oh, s_ref[...], preferred_element_type=jnp.float32) * h
             + jnp.dot(oh, bi_ref[...], preferred_element_type=jnp.float32))
        xy = xy_ref[...]
        pad = jnp.zeros((_BN, _WF_M - _FM - 3), jnp.float32)
        out_ref[...] = jnp.concatenate([h, xy, pad], axis=1)
        tpad = jnp.zeros((_BN, _WT_M - _FM // 2 - 3), jnp.float32)
        tab_ref[...] = jnp.concatenate([_pack_bf16(h), xy, tpad], axis=1)

    return pl.pallas_call(
        body,
        grid=(grid,),
        in_specs=[
            pl.BlockSpec((_BN, x2.shape[1]), lambda i: (i, 0)),
            pl.BlockSpec((_BN, 1), lambda i: (i, 0)),
            pl.BlockSpec((_BN, 3), lambda i: (i, 0)),
            _full_spec(t2.shape),
            _full_spec(s0.shape),
            _full_spec(b0.shape),
            _full_spec(wne.shape),
            _full_spec(bne.shape),
        ],
        out_specs=[pl.BlockSpec((_BN, _WF_M), lambda i: (i, 0)),
                   pl.BlockSpec((_BN, _WT_M), lambda i: (i, 0))],
        out_shape=[jax.ShapeDtypeStruct((_NP, _WF_M), jnp.float32),
                   jax.ShapeDtypeStruct((_NP, _WT_M), jnp.float32)],
    )(x2, batch2, coord2, t2, s0, b0, wne, bne)


def _cond_final_call(hcx, batch2, wco, bco, wfc, bfc):
    def body(hx_ref, b_ref, wco_ref, bco_ref, wfc_ref, bfc_ref, out_ref):
        hc = hx_ref[...][:, :_FC]
        h = jnp.dot(hc, wco_ref[...],
                    preferred_element_type=jnp.float32) + bco_ref[...]
        oh = (b_ref[...] == lax.broadcasted_iota(jnp.int32, (_NP, _G), 1)
              ).astype(jnp.float32)
        seg = lax.dot_general(oh, h, (((0,), (0,)), ((), ())),
                              preferred_element_type=jnp.float32)
        ones = jnp.ones((_NP, 1), jnp.float32)
        cnt = lax.dot_general(oh, ones, (((0,), (0,)), ((), ())),
                              preferred_element_type=jnp.float32)
        g = seg / jnp.maximum(cnt, 1.0)
        out_ref[...] = jnp.dot(g, wfc_ref[...],
                               preferred_element_type=jnp.float32) + bfc_ref[...]

    return pl.pallas_call(
        body,
        grid=(1,),
        in_specs=[_full_spec(hcx.shape), _full_spec(batch2.shape),
                  _full_spec(wco.shape), _full_spec(bco.shape),
                  _full_spec(wfc.shape), _full_spec(bfc.shape)],
        out_specs=_full_spec((_G, wfc.shape[1])),
        out_shape=jax.ShapeDtypeStruct((_G, wfc.shape[1]), jnp.float32),
    )(hcx, batch2, wco, bco, wfc, bfc)


def _pred_call(hx, w1, b1, w2, b2, w3, b3):
    grid = _NP // _BN

    def body(hx_ref, w1_ref, b1_ref, w2_ref, b2_ref, w3_ref, b3_ref, out_ref):
        h = hx_ref[...][:, :_FM]
        o = jax.nn.relu(jnp.dot(h, w1_ref[...],
                                preferred_element_type=jnp.float32) + b1_ref[...])
        o = jax.nn.relu(jnp.dot(o, w2_ref[...],
                                preferred_element_type=jnp.float32) + b2_ref[...])
        out_ref[...] = jnp.dot(o, w3_ref[...],
                               preferred_element_type=jnp.float32) + b3_ref[...]

    return pl.pallas_call(
        body,
        grid=(grid,),
        in_specs=[
            pl.BlockSpec((_BN, _WF_M), lambda i: (i, 0)),
            _full_spec(w1.shape), _full_spec(b1.shape),
            _full_spec(w2.shape), _full_spec(b2.shape),
            _full_spec(w3.shape), _full_spec(b3.shape),
        ],
        out_specs=pl.BlockSpec((_BN, w3.shape[1]), lambda i: (i, 0)),
        out_shape=jax.ShapeDtypeStruct((_NP, w3.shape[1]), jnp.float32),
    )(hx, w1, b1, w2, b2, w3, b3)


def _gcl_edge_phase(tab, rowi, coli, ea2, zeros_np, w, normalize, f):
    hrow, hcol = _sc_gather(tab, rowi, coli)
    mt = _edge_call(hrow, hcol, ea2,
                    w["w1a"], w["w1b"], w["wr3"], w["w1e"], w["b1f"],
                    w["w2"], w["b2"], w["wc1"], w["bc1"], w["wc2"],
                    normalize=normalize, f=f)
    return _sc_scatter(mt, rowi, zeros_np)


def _fold_gcl(p, f, wf, we=None, be=None):
    bf16 = jnp.bfloat16
    w1 = p["edge_mlp"][0]["W"]
    b1 = p["edge_mlp"][0]["b"]
    fp = f // 2
    w1a = jnp.stack([w1[:fp], w1[fp:f]]).astype(bf16)
    w1b = jnp.stack([w1[f:f + fp], w1[f + fp:2 * f]]).astype(bf16)
    w1r = w1[2 * f:2 * f + 1]
    if we is None:
        w1e = w1[2 * f + 1:2 * f + 2]
        b1f = b1[None, :]
    else:
        w1tail = w1[2 * f + 1:]
        w1e = we @ w1tail
        b1f = (b1 + be @ w1tail)[None, :]
    wr3 = jnp.concatenate([w1r, w1r, w1r], axis=0)
    w2 = p["edge_mlp"][1]["W"].astype(bf16)
    b2 = p["edge_mlp"][1]["b"][None, :]
    wc1 = p["coord_mlp"][0]["W"].astype(bf16)
    bc1 = p["coord_mlp"][0]["b"][None, :]
    wc2 = p["coord_mlp"][1]["W"].astype(bf16)
    wn1 = p["node_mlp"][0]["W"]
    w1h = wn1[:f]
    w1ag = wn1[f:]
    bn1 = p["node_mlp"][0]["b"][None, :]
    wn2 = p["node_mlp"][1]["W"]
    bn2 = p["node_mlp"][1]["b"][None, :]
    return dict(w1a=w1a, w1b=w1b, wr3=wr3, w1e=w1e, b1f=b1f, w2=w2, b2=b2,
                wc1=wc1, bc1=bc1, wc2=wc2, w1h=w1h, w1ag=w1ag, bn1=bn1,
                wn2=wn2, bn2=bn2)


def kernel(x, edge_index, edge_attr, x_coord, cond, timesteps, batch, params):
    f32 = jnp.float32
    i32 = jnp.int32

    x2 = jnp.zeros((_NP, _PRED_H * _NODE_FEAT), f32).at[:_N].set(
        x.reshape(_N, -1))
    loops = jnp.arange(_N, dtype=i32)
    row = jnp.concatenate([edge_index[0].astype(i32), loops])
    col = jnp.concatenate([edge_index[1].astype(i32), loops])
    ea = jnp.concatenate([edge_attr.astype(f32), jnp.zeros((_N,), f32)])
    rowi = jnp.full((_EP,), _NP - 1, i32).at[:_E].set(row).reshape(
        _NC * _NS, _CH, 128)
    coli = jnp.full((_EP,), _NP - 1, i32).at[:_E].set(col).reshape(
        _NC * _NS, _CH, 128)
    ea2 = jnp.zeros((_EP, 1), f32).at[:_E, 0].set(ea)
    batch2 = jnp.full((_NP, 1), _G, i32).at[:_N, 0].set(batch.astype(i32))
    coord2 = jnp.zeros((_NP, 3), f32).at[:_N].set(x_coord.astype(f32))
    cond2 = jnp.zeros((_NP, cond.shape[1]), f32).at[:_N].set(cond)
    zeros_np = jnp.zeros((_NP, _MT_W), f32)
    ts2 = timesteps.astype(i32).reshape(_G, 1)

    t2 = _t_call(ts2, _pe_tab(),
                 params["dse1"]["W"], params["dse1"]["b"][None, :],
                 params["dse2"]["W"], params["dse2"]["b"][None, :])

    hcx, tabc = _cond_init_call(cond2, coord2, params["cond_emb_in"]["W"],
                                params["cond_emb_in"]["b"][None, :])
    for p in params["cond_gcl"]:
        w = _fold_gcl(p, _FC, _WF_C)
        acc2 = _gcl_edge_phase(tabc, rowi, coli, ea2, zeros_np, w,
                               normalize=False, f=_FC)
        dummy = jnp.zeros((_G, _FC), f32)
        hcx, tabc = _node_call(hcx, acc2, batch2, w["w1h"],
                               w["w1ag"], w["bn1"], w["wn2"], w["bn2"],
                               dummy, dummy, film=False, f=_FC)

    emb = _cond_final_call(hcx, batch2, params["cond_emb_out"]["W"],
                           params["cond_emb_out"]["b"][None, :],
                           params["cond_fc"]["W"],
                           params["cond_fc"]["b"][None, :])
    er = emb.reshape(5, _G, 2, _FM)
    scl = er[:, :, 0]
    bia = er[:, :, 1]

    hx, tabm = _main_init_call(x2, batch2, coord2, t2, scl[0], bia[0],
                               params["node_emb"]["W"],
                               params["node_emb"]["b"][None, :])
    we = params["edge_emb"]["W"]
    be = params["edge_emb"]["b"]
    for l in range(5):
        w = _fold_gcl(params["layers"][l], _FM, _WF_M, we=we, be=be)
        acc2 = _gcl_edge_phase(tabm, rowi, coli, ea2, zeros_np, w,
                               normalize=True, f=_FM)
        film = l < 4
        s_l = scl[l + 1] if film else jnp.zeros((_G, _FM), f32)
        b_l = bia[l + 1] if film else jnp.zeros((_G, _FM), f32)
        hx, tabm = _node_call(hx, acc2, batch2, w["w1h"],
                              w["w1ag"], w["bn1"], w["wn2"], w["bn2"],
                              s_l, b_l, film=film, f=_FM)

    pred = _pred_call(hx, params["pred1"]["W"], params["pred1"]["b"][None, :],
                      params["pred2"]["W"], params["pred2"]["b"][None, :],
                      params["pred3"]["W"], params["pred3"]["b"][None, :])
    node_pred = pred[:_N].reshape(_N, _PRED_H, _NODE_FEAT)
    x_v = hx[:_N, _FM:_FM + 3]
    return node_pred, x_v


# speedup vs baseline: 1.6249x; 1.1223x over previous
import functools

import jax
import jax.numpy as jnp
import numpy as np
from jax import lax
from jax.experimental import pallas as pl
from jax.experimental.pallas import tpu as pltpu
from jax.experimental.pallas import tpu_sc as plsc

_N = 10000
_E_RAW = 320000
_E = _E_RAW + _N
_G = 16
_H = 64
_DSE = 32
_FM = _H + _DSE
_FC = _H
_PRED_H = 16
_NODE_FEAT = 4
_N_DIFF = 200

_NP = 10240
_EP = 331776
_EPH = _EP
_WF_C = 80
_WF_M = 112
_WT_C = 48
_WT_M = 64
_MT_W = 80

_BE = 2048
_BN = 1024

_NC, _NS = 2, 16
_CH = _EPH // 128 // (_NC * _NS)
_NROW = _NP // _NS


def _pe_tab():
    pos = np.arange(_N_DIFF, dtype=np.float64)[:, None] + 1.0
    div = np.exp(np.arange(0, _DSE, 2, dtype=np.float64) * -(np.log(10000.0) / _DSE))
    pe = np.zeros((_N_DIFF, _DSE), dtype=np.float32)
    pe[:, 0::2] = np.sin(pos * div)
    pe[:, 1::2] = np.cos(pos * div)
    return jnp.asarray(pe)


def _silu(v):
    return v * (0.5 * jnp.tanh(0.5 * v) + 0.5)


def _full_spec(shape):
    nd = len(shape)
    return pl.BlockSpec(shape, lambda i: (0,) * nd)


def _sc_mesh():
    return plsc.VectorSubcoreMesh(core_axis_name="c", subcore_axis_name="s",
                                  num_cores=_NC, num_subcores=_NS)


_SC_PARAMS = pltpu.CompilerParams(use_tc_tiling_on_sc=False)


def _sc_gather(table, rowi, coli):
    w = table.shape[1]
    nb = 3

    def body(tab, ri, ci, outr, outc, idxr, idxc, tabs,
             bufr0, bufr1, bufr2, bufc0, bufc1, bufc2,
             sgr0, sgr1, sgr2, sgc0, sgc1, sgc2,
             swr0, swr1, swr2, swc0, swc1, swc2):
        bufr = (bufr0, bufr1, bufr2)
        bufc = (bufc0, bufc1, bufc2)
        sgr = (sgr0, sgr1, sgr2)
        sgc = (sgc0, sgc1, sgc2)
        swr = (swr0, swr1, swr2)
        swc = (swc0, swc1, swc2)
        c = lax.axis_index("c")
        s = lax.axis_index("s")
        wid = s * _NC + c
        base = wid * _CH
        pltpu.sync_copy(tab.at[pl.ds(s * _NROW, _NROW)],
                        tabs.at[pl.ds(s * _NROW, _NROW)])
        pltpu.sync_copy(ri.at[wid], idxr)
        pltpu.sync_copy(ci.at[wid], idxc)
        plsc.subcore_barrier()

        def wr_desc(b, j):
            return (pltpu.make_async_copy(
                        bufr[b], outr.at[pl.ds((base + j) * 128, 128)], swr[b]),
                    pltpu.make_async_copy(
                        bufc[b], outc.at[pl.ds((base + j) * 128, 128)], swc[b]))

        def gd_desc(b, j):
            return (pltpu.make_async_copy(tabs.at[idxr.at[j]], bufr[b], sgr[b]),
                    pltpu.make_async_copy(tabs.at[idxc.at[j]], bufc[b], sgc[b]))

        def group(g, carry):
            for b in range(nb):
                j = g * nb + b
                pb = (b + nb - 1) % nb

                @pl.when(j >= nb)
                def _():
                    d1, d2 = wr_desc(b, j - nb)
                    d1.wait()
                    d2.wait()

                g1, g2 = gd_desc(b, j)
                g1.start()
                g2.start()

                @pl.when(j >= 1)
                def _():
                    p1, p2 = gd_desc(pb, j - 1)
                    p1.wait()
                    p2.wait()
                    e1, e2 = wr_desc(pb, j - 1)
                    e1.start()
                    e2.start()
            return carry

        lax.fori_loop(0, _CH // nb, group, 0)
        last = _CH - 1
        lb = last % nb
        p1, p2 = gd_desc(lb, last)
        p1.wait()
        p2.wait()
        e1, e2 = wr_desc(lb, last)
        e1.start()
        e2.start()
        for b in range(nb):
            j = _CH - nb + b
            d1, d2 = wr_desc(j % nb, j)
            d1.wait()
            d2.wait()

    return pl.kernel(
        body,
        out_type=[jax.ShapeDtypeStruct((_EPH, w), jnp.float32)] * 2,
        mesh=_sc_mesh(),
        scratch_types=(
            [pltpu.VMEM((_CH, 128), jnp.int32)] * 2
            + [pltpu.VMEM_SHARED((_NP, w), jnp.float32)]
            + [pltpu.VMEM((128, w), jnp.float32)] * 6
            + [pltpu.SemaphoreType.DMA] * 12
        ),
        compiler_params=_SC_PARAMS,
    )(table, rowi, coli)


def _sc_scatter(mt, rowi, zeros_np):
    nb = 3

    def body(mt_h, ri, z_h, out_h, idx, buf0, buf1, buf2, acc_sh,
             sl0, sl1, sl2, sa0, sa1, sa2):
        buf = (buf0, buf1, buf2)
        sl = (sl0, sl1, sl2)
        sa = (sa0, sa1, sa2)
        c = lax.axis_index("c")
        s = lax.axis_index("s")
        wid = s * _NC + c
        base = wid * _CH
        pltpu.sync_copy(z_h.at[pl.ds(s * _NROW, _NROW)],
                        acc_sh.at[pl.ds(s * _NROW, _NROW)])
        pltpu.sync_copy(ri.at[wid], idx)
        plsc.subcore_barrier()

        def ld_desc(b, j):
            return pltpu.make_async_copy(
                mt_h.at[pl.ds((base + j) * 128, 128)], buf[b], sl[b])

        def add_desc(b, j):
            return pltpu.make_async_copy(buf[b], acc_sh.at[idx.at[j]], sa[b])

        def group(g, carry):
            for b in range(nb):
                j = g * nb + b
                pb = (b + nb - 1) % nb

                @pl.when(j >= nb)
                def _():
                    add_desc(b, j - nb).wait()

                ld_desc(b, j).start()

                @pl.when(j >= 1)
                def _():
                    ld_desc(pb, j - 1).wait()
                    pltpu.async_copy(buf[pb], acc_sh.at[idx.at[j - 1]],
                                     sa[pb], add=True)
            return carry

        lax.fori_loop(0, _CH // nb, group, 0)
        last = _CH - 1
        lb = last % nb
        ld_desc(lb, last).wait()
        pltpu.async_copy(buf[lb], acc_sh.at[idx.at[last]], sa[lb], add=True)
        for b in range(nb):
            j = _CH - nb + b
            add_desc(j % nb, j).wait()
        plsc.subcore_barrier()
        pltpu.sync_copy(acc_sh.at[pl.ds(s * _NROW, _NROW)],
                        out_h.at[c, pl.ds(s * _NROW, _NROW)])

    return pl.kernel(
        body,
        out_type=jax.ShapeDtypeStruct((2, _NP, _MT_W), jnp.float32),
        mesh=_sc_mesh(),
        scratch_types=(
            [pltpu.VMEM((_CH, 128), jnp.int32)]
            + [pltpu.VMEM((128, _MT_W), jnp.float32)] * 3
            + [pltpu.VMEM_SHARED((_NP, _MT_W), jnp.float32)]
            + [pltpu.SemaphoreType.DMA] * 6
        ),
        compiler_params=_SC_PARAMS,
    )(mt, rowi, zeros_np)


def _unpack2_bf16(v):
    uw = jax.lax.bitcast_convert_type(v, jnp.uint32)
    a = jax.lax.bitcast_convert_type(uw & jnp.uint32(0xFFFF0000), jnp.float32)
    b = jax.lax.bitcast_convert_type(uw << 16, jnp.float32)
    return a.astype(jnp.bfloat16), b.astype(jnp.bfloat16)


def _pack_bf16(v):
    h = v.shape[1] // 2
    r1 = v[:, :h].astype(jnp.bfloat16).astype(jnp.float32)
    r2 = v[:, h:].astype(jnp.bfloat16).astype(jnp.float32)
    u1 = jax.lax.bitcast_convert_type(r1, jnp.uint32)
    u2 = jax.lax.bitcast_convert_type(r2, jnp.uint32)
    w = (u1 & jnp.uint32(0xFFFF0000)) | (u2 >> 16)
    return jax.lax.bitcast_convert_type(w, jnp.float32)


def _edge_call(hrow, hcol, ea2, w1a, w1b, wr3, w1e, b1f, w2, b2, wc1, bc1,
               wc2, normalize, f):
    wf = hrow.shape[1]
    fp = f // 2
    grid = _EPH // _BE

    def body(hr_ref, hc_ref, ea_ref, w1a_ref, w1b_ref, wr3_ref, w1e_ref,
             b1f_ref, w2_ref, b2_ref, wc1_ref, bc1_ref, wc2_ref, out_ref):
        hr = hr_ref[...]
        hc = hc_ref[...]
        cd = hr[:, fp:fp + 3] - hc[:, fp:fp + 3]
        cdsq = cd * cd
        ra, rb = _unpack2_bf16(hr[:, :fp])
        ca, cb = _unpack2_bf16(hc[:, :fp])
        pre = (jnp.dot(ra, w1a_ref[0], preferred_element_type=jnp.float32)
               + jnp.dot(rb, w1a_ref[1], preferred_element_type=jnp.float32)
               + jnp.dot(ca, w1b_ref[0], preferred_element_type=jnp.float32)
               + jnp.dot(cb, w1b_ref[1], preferred_element_type=jnp.float32))
        radial = jnp.sum(cdsq, axis=1, keepdims=True)
        pre = (pre + radial * wr3_ref[0:1] + ea_ref[...] * w1e_ref[...]
               + b1f_ref[...])
        h1 = _silu(pre)
        m = _silu(jnp.dot(h1.astype(jnp.bfloat16), w2_ref[...],
                          preferred_element_type=jnp.float32) + b2_ref[...])
        c1 = _silu(jnp.dot(m.astype(jnp.bfloat16), wc1_ref[...],
                           preferred_element_type=jnp.float32) + bc1_ref[...])
        cm = jnp.dot(c1.astype(jnp.bfloat16), wc2_ref[...],
                     preferred_element_type=jnp.float32)
        if normalize:
            sc_ = cm / (jnp.sqrt(radial) + 1e-8)
        else:
            sc_ = cm
        ones = jnp.ones((_BE, 1), jnp.float32)
        zer = jnp.zeros((_BE, _MT_W - _H - 4), jnp.float32)
        out_ref[...] = jnp.concatenate([m, cd * sc_, ones, zer], axis=1)

    return pl.pallas_call(
        body,
        grid=(grid,),
        in_specs=[
            pl.BlockSpec((_BE, wf), lambda i: (i, 0)),
            pl.BlockSpec((_BE, wf), lambda i: (i, 0)),
            pl.BlockSpec((_BE, 1), lambda i: (i, 0)),
            _full_spec(w1a.shape),
            _full_spec(w1b.shape),
            _full_spec(wr3.shape),
            _full_spec(w1e.shape),
            _full_spec(b1f.shape),
            _full_spec(w2.shape),
            _full_spec(b2.shape),
            _full_spec(wc1.shape),
            _full_spec(bc1.shape),
            _full_spec(wc2.shape),
        ],
        out_specs=pl.BlockSpec((_BE, _MT_W), lambda i: (i, 0)),
        out_shape=jax.ShapeDtypeStruct((_EPH, _MT_W), jnp.float32),
    )(hrow, hcol, ea2, w1a, w1b, wr3, w1e, b1f, w2, b2, wc1, bc1, wc2)


def _node_call(hx, acca, batch2, w1h, w1a, b1, w2, b2, scales, biases,
               film, f):
    wf = hx.shape[1]
    wt = _WT_M if f == _FM else _WT_C
    grid = _NP // _BN

    def body(hx_ref, acc_ref, b_ref, w1h_ref, w1a_ref, b1_ref,
             w2_ref, b2_ref, s_ref, bi_ref, out_ref, tab_ref):
        hxv = hx_ref[...]
        h = hxv[:, :f]
        coord = hxv[:, f:f + 3]
        acc = acc_ref[0] + acc_ref[1]
        agg = acc[:, :_H]
        tr = acc[:, _H:_H + 3]
        cnt = acc[:, _H + 3:_H + 4]
        coord2 = coord + tr / jnp.maximum(cnt, 1.0)
        pre = (jnp.dot(h, w1h_ref[...], preferred_element_type=jnp.float32)
               + jnp.dot(agg, w1a_ref[...], preferred_element_type=jnp.float32)
               + b1_ref[...])
        hmid = _silu(pre)
        h2 = h + jnp.dot(hmid, w2_ref[...],
                         preferred_element_type=jnp.float32) + b2_ref[...]
        if film:
            bb = b_ref[...]
            oh = (bb == lax.broadcasted_iota(jnp.int32, (_BN, _G), 1)
                  ).astype(jnp.float32)
            h2 = (jnp.dot(oh, s_ref[...], preferred_element_type=jnp.float32)
                  * h2
                  + jnp.dot(oh, bi_ref[...], preferred_element_type=jnp.float32))
        pad = jnp.zeros((_BN, wf - f - 3), jnp.float32)
        out_ref[...] = jnp.concatenate([h2, coord2, pad], axis=1)
        tpad = jnp.zeros((_BN, wt - f // 2 - 3), jnp.float32)
        tab_ref[...] = jnp.concatenate([_pack_bf16(h2), coord2, tpad], axis=1)

    return pl.pallas_call(
        body,
        grid=(grid,),
        in_specs=[
            pl.BlockSpec((_BN, wf), lambda i: (i, 0)),
            pl.BlockSpec((2, _BN, _MT_W), lambda i: (0, i, 0)),
            pl.BlockSpec((_BN, 1), lambda i: (i, 0)),
            _full_spec(w1h.shape),
            _full_spec(w1a.shape),
            _full_spec(b1.shape),
            _full_spec(w2.shape),
            _full_spec(b2.shape),
            _full_spec(scales.shape),
            _full_spec(biases.shape),
        ],
        out_specs=[pl.BlockSpec((_BN, wf), lambda i: (i, 0)),
                   pl.BlockSpec((_BN, wt), lambda i: (i, 0))],
        out_shape=[jax.ShapeDtypeStruct((_NP, wf), jnp.float32),
                   jax.ShapeDtypeStruct((_NP, wt), jnp.float32)],
    )(hx, acca, batch2, w1h, w1a, b1, w2, b2, scales, biases)


def _cond_init_call(cond2, coord2, wci, bci):
    grid = _NP // _BN

    def body(c_ref, xy_ref, w_ref, b_ref, out_ref, tab_ref):
        h = jnp.dot(c_ref[...], w_ref[...],
                    preferred_element_type=jnp.float32) + b_ref[...]
        xy = xy_ref[...]
        pad = jnp.zeros((_BN, _WF_C - _FC - 3), jnp.float32)
        out_ref[...] = jnp.concatenate([h, xy, pad], axis=1)
        tpad = jnp.zeros((_BN, _WT_C - _FC // 2 - 3), jnp.float32)
        tab_ref[...] = jnp.concatenate([_pack_bf16(h), xy, tpad], axis=1)

    return pl.pallas_call(
        body,
        grid=(grid,),
        in_specs=[
            pl.BlockSpec((_BN, cond2.shape[1]), lambda i: (i, 0)),
            pl.BlockSpec((_BN, 3), lambda i: (i, 0)),
            _full_spec(wci.shape),
            _full_spec(bci.shape),
        ],
        out_specs=[pl.BlockSpec((_BN, _WF_C), lambda i: (i, 0)),
                   pl.BlockSpec((_BN, _WT_C), lambda i: (i, 0))],
        out_shape=[jax.ShapeDtypeStruct((_NP, _WF_C), jnp.float32),
                   jax.ShapeDtypeStruct((_NP, _WT_C), jnp.float32)],
    )(cond2, coord2, wci, bci)


def _t_call(ts2, pe, wd1, bd1, wd2, bd2):
    def body(ts_ref, pe_ref, w1_ref, b1_ref, w2_ref, b2_ref, out_ref):
        oh = (ts_ref[...] == lax.broadcasted_iota(jnp.int32, (_G, _N_DIFF), 1)
              ).astype(jnp.float32)
        t = jnp.dot(oh, pe_ref[...], preferred_element_type=jnp.float32)
        v = jnp.dot(t, w1_ref[...], preferred_element_type=jnp.float32) + b1_ref[...]
        u = v * jnp.tanh(jax.nn.softplus(v))
        out_ref[...] = jnp.dot(u, w2_ref[...],
                               preferred_element_type=jnp.float32) + b2_ref[...]

    return pl.pallas_call(
        body,
        grid=(1,),
        in_specs=[_full_spec(ts2.shape), _full_spec(pe.shape),
                  _full_spec(wd1.shape), _full_spec(bd1.shape),
                  _full_spec(wd2.shape), _full_spec(bd2.shape)],
        out_specs=_full_spec((_G, _DSE)),
        out_shape=jax.ShapeDtypeStruct((_G, _DSE), jnp.float32),
    )(ts2, pe, wd1, bd1, wd2, bd2)


def _main_init_call(x2, batch2, coord2, t2, s0, b0, wne, bne):
    grid = _NP // _BN

    def body(x_ref, b_ref, xy_ref, t_ref, s_ref, bi_ref, w_ref, bn_ref,
             out_ref, tab_ref):
        oh = (b_ref[...] == lax.broadcasted_iota(jnp.int32, (_BN, _G), 1)
              ).astype(jnp.float32)
        temb = jnp.dot(oh, t_ref[...], preferred_element_type=jnp.float32)
        hn = jnp.dot(x_ref[...], w_ref[...],
                     preferred_element_type=jnp.float32) + bn_ref[...]
        h = jnp.concatenate([hn, temb], axis=1)
        h = (jnp.dot(oh, s_ref[...], preferred_element_type=jnp.float32) * h
             + jnp.dot(oh, bi_ref[...], preferred_element_type=jnp.float32))
        xy = xy_ref[...]
        pad = jnp.zeros((_BN, _WF_M - _FM - 3), jnp.float32)
        out_ref[...] = jnp.concatenate([h, xy, pad], axis=1)
        tpad = jnp.zeros((_BN, _WT_M - _FM // 2 - 3), jnp.float32)
        tab_ref[...] = jnp.concatenate([_pack_bf16(h), xy, tpad], axis=1)

    return pl.pallas_call(
        body,
        grid=(grid,),
        in_specs=[
            pl.BlockSpec((_BN, x2.shape[1]), lambda i: (i, 0)),
            pl.BlockSpec((_BN, 1), lambda i: (i, 0)),
            pl.BlockSpec((_BN, 3), lambda i: (i, 0)),
            _full_spec(t2.shape),
            _full_spec(s0.shape),
            _full_spec(b0.shape),
            _full_spec(wne.shape),
            _full_spec(bne.shape),
        ],
        out_specs=[pl.BlockSpec((_BN, _WF_M), lambda i: (i, 0)),
                   pl.BlockSpec((_BN, _WT_M), lambda i: (i, 0))],
        out_shape=[jax.ShapeDtypeStruct((_NP, _WF_M), jnp.float32),
                   jax.ShapeDtypeStruct((_NP, _WT_M), jnp.float32)],
    )(x2, batch2, coord2, t2, s0, b0, wne, bne)


def _cond_final_call(hcx, batch2, wco, bco, wfc, bfc):
    def body(hx_ref, b_ref, wco_ref, bco_ref, wfc_ref, bfc_ref, out_ref):
        hc = hx_ref[...][:, :_FC]
        h = jnp.dot(hc, wco_ref[...],
                    preferred_element_type=jnp.float32) + bco_ref[...]
        oh = (b_ref[...] == lax.broadcasted_iota(jnp.int32, (_NP, _G), 1)
              ).astype(jnp.float32)
        seg = lax.dot_general(oh, h, (((0,), (0,)), ((), ())),
                              preferred_element_type=jnp.float32)
        ones = jnp.ones((_NP, 1), jnp.float32)
        cnt = lax.dot_general(oh, ones, (((0,), (0,)), ((), ())),
                              preferred_element_type=jnp.float32)
        g = seg / jnp.maximum(cnt, 1.0)
        out_ref[...] = jnp.dot(g, wfc_ref[...],
                               preferred_element_type=jnp.float32) + bfc_ref[...]

    return pl.pallas_call(
        body,
        grid=(1,),
        in_specs=[_full_spec(hcx.shape), _full_spec(batch2.shape),
                  _full_spec(wco.shape), _full_spec(bco.shape),
                  _full_spec(wfc.shape), _full_spec(bfc.shape)],
        out_specs=_full_spec((_G, wfc.shape[1])),
        out_shape=jax.ShapeDtypeStruct((_G, wfc.shape[1]), jnp.float32),
    )(hcx, batch2, wco, bco, wfc, bfc)


def _pred_call(hx, w1, b1, w2, b2, w3, b3):
    grid = _NP // _BN

    def body(hx_ref, w1_ref, b1_ref, w2_ref, b2_ref, w3_ref, b3_ref, out_ref):
        h = hx_ref[...][:, :_FM]
        o = jax.nn.relu(jnp.dot(h, w1_ref[...],
                                preferred_element_type=jnp.float32) + b1_ref[...])
        o = jax.nn.relu(jnp.dot(o, w2_ref[...],
                                preferred_element_type=jnp.float32) + b2_ref[...])
        out_ref[...] = jnp.dot(o, w3_ref[...],
                               preferred_element_type=jnp.float32) + b3_ref[...]

    return pl.pallas_call(
        body,
        grid=(grid,),
        in_specs=[
            pl.BlockSpec((_BN, _WF_M), lambda i: (i, 0)),
            _full_spec(w1.shape), _full_spec(b1.shape),
            _full_spec(w2.shape), _full_spec(b2.shape),
            _full_spec(w3.shape), _full_spec(b3.shape),
        ],
        out_specs=pl.BlockSpec((_BN, w3.shape[1]), lambda i: (i, 0)),
        out_shape=jax.ShapeDtypeStruct((_NP, w3.shape[1]), jnp.float32),
    )(hx, w1, b1, w2, b2, w3, b3)


def _gcl_edge_phase(tab, rowi, coli, ea2, zeros_np, w, normalize, f):
    hrow, hcol = _sc_gather(tab, rowi, coli)
    mt = _edge_call(hrow, hcol, ea2,
                    w["w1a"], w["w1b"], w["wr3"], w["w1e"], w["b1f"],
                    w["w2"], w["b2"], w["wc1"], w["bc1"], w["wc2"],
                    normalize=normalize, f=f)
    return _sc_scatter(mt, rowi, zeros_np)


def _fold_gcl(p, f, wf, we=None, be=None):
    bf16 = jnp.bfloat16
    w1 = p["edge_mlp"][0]["W"]
    b1 = p["edge_mlp"][0]["b"]
    fp = f // 2
    w1a = jnp.stack([w1[:fp], w1[fp:f]]).astype(bf16)
    w1b = jnp.stack([w1[f:f + fp], w1[f + fp:2 * f]]).astype(bf16)
    w1r = w1[2 * f:2 * f + 1]
    if we is None:
        w1e = w1[2 * f + 1:2 * f + 2]
        b1f = b1[None, :]
    else:
        w1tail = w1[2 * f + 1:]
        w1e = we @ w1tail
        b1f = (b1 + be @ w1tail)[None, :]
    wr3 = jnp.concatenate([w1r, w1r, w1r], axis=0)
    w2 = p["edge_mlp"][1]["W"].astype(bf16)
    b2 = p["edge_mlp"][1]["b"][None, :]
    wc1 = p["coord_mlp"][0]["W"].astype(bf16)
    bc1 = p["coord_mlp"][0]["b"][None, :]
    wc2 = p["coord_mlp"][1]["W"].astype(bf16)
    wn1 = p["node_mlp"][0]["W"]
    w1h = wn1[:f]
    w1ag = wn1[f:]
    bn1 = p["node_mlp"][0]["b"][None, :]
    wn2 = p["node_mlp"][1]["W"]
    bn2 = p["node_mlp"][1]["b"][None, :]
    return dict(w1a=w1a, w1b=w1b, wr3=wr3, w1e=w1e, b1f=b1f, w2=w2, b2=b2,
                wc1=wc1, bc1=bc1, wc2=wc2, w1h=w1h, w1ag=w1ag, bn1=bn1,
                wn2=wn2, bn2=bn2)


def kernel(x, edge_index, edge_attr, x_coord, cond, timesteps, batch, params):
    f32 = jnp.float32
    i32 = jnp.int32

    x2 = jnp.zeros((_NP, _PRED_H * _NODE_FEAT), f32).at[:_N].set(
        x.reshape(_N, -1))
    loops = jnp.arange(_N, dtype=i32)
    row = jnp.concatenate([edge_index[0].astype(i32), loops])
    col = jnp.concatenate([edge_index[1].astype(i32), loops])
    ea = jnp.concatenate([edge_attr.astype(f32), jnp.zeros((_N,), f32)])
    rowi = jnp.full((_EP,), _NP - 1, i32).at[:_E].set(row).reshape(
        _NC * _NS, _CH, 128)
    coli = jnp.full((_EP,), _NP - 1, i32).at[:_E].set(col).reshape(
        _NC * _NS, _CH, 128)
    ea2 = jnp.zeros((_EP, 1), f32).at[:_E, 0].set(ea)
    batch2 = jnp.full((_NP, 1), _G, i32).at[:_N, 0].set(batch.astype(i32))
    coord2 = jnp.zeros((_NP, 3), f32).at[:_N].set(x_coord.astype(f32))
    cond2 = jnp.zeros((_NP, cond.shape[1]), f32).at[:_N].set(cond)
    zeros_np = jnp.zeros((_NP, _MT_W), f32)
    ts2 = timesteps.astype(i32).reshape(_G, 1)

    t2 = _t_call(ts2, _pe_tab(),
                 params["dse1"]["W"], params["dse1"]["b"][None, :],
                 params["dse2"]["W"], params["dse2"]["b"][None, :])

    hcx, tabc = _cond_init_call(cond2, coord2, params["cond_emb_in"]["W"],
                                params["cond_emb_in"]["b"][None, :])
    for p in params["cond_gcl"]:
        w = _fold_gcl(p, _FC, _WF_C)
        acc2 = _gcl_edge_phase(tabc, rowi, coli, ea2, zeros_np, w,
                               normalize=False, f=_FC)
        dummy = jnp.zeros((_G, _FC), f32)
        hcx, tabc = _node_call(hcx, acc2, batch2, w["w1h"],
                               w["w1ag"], w["bn1"], w["wn2"], w["bn2"],
                               dummy, dummy, film=False, f=_FC)

    emb = _cond_final_call(hcx, batch2, params["cond_emb_out"]["W"],
                           params["cond_emb_out"]["b"][None, :],
                           params["cond_fc"]["W"],
                           params["cond_fc"]["b"][None, :])
    er = emb.reshape(5, _G, 2, _FM)
    scl = er[:, :, 0]
    bia = er[:, :, 1]

    hx, tabm = _main_init_call(x2, batch2, coord2, t2, scl[0], bia[0],
                               params["node_emb"]["W"],
                               params["node_emb"]["b"][None, :])
    we = params["edge_emb"]["W"]
    be = params["edge_emb"]["b"]
    for l in range(5):
        w = _fold_gcl(params["layers"][l], _FM, _WF_M, we=we, be=be)
        acc2 = _gcl_edge_phase(tabm, rowi, coli, ea2, zeros_np, w,
                               normalize=True, f=_FM)
        film = l < 4
        s_l = scl[l + 1] if film else jnp.zeros((_G, _FM), f32)
        b_l = bia[l + 1] if film else jnp.zeros((_G, _FM), f32)
        hx, tabm = _node_call(hx, acc2, batch2, w["w1h"],
                              w["w1ag"], w["bn1"], w["wn2"], w["bn2"],
                              s_l, b_l, film=film, f=_FM)

    pred = _pred_call(hx, params["pred1"]["W"], params["pred1"]["b"][None, :],
                      params["pred2"]["W"], params["pred2"]["b"][None, :],
                      params["pred3"]["W"], params["pred3"]["b"][None, :])
    node_pred = pred[:_N].reshape(_N, _PRED_H, _NODE_FEAT)
    x_v = hx[:_N, _FM:_FM + 3]
    return node_pred, x_v
